# baseline (speedup 1.0000x reference)
import functools

import jax
import jax.numpy as jnp
from jax import lax
from jax.experimental import pallas as pl
from jax.experimental.pallas import tpu as pltpu
from jax.experimental.pallas import tpu_sc as plsc

B = 16384
D = 128
L = 16
NC = 2
NS = 16
NW = NC * NS
ROWS_PER_W = B // NW
CHUNK = 128
NCHUNK = ROWS_PER_W // CHUNK
NBUF = 2


def _rsqrt_newton(x):
    i = lax.bitcast_convert_type(x, jnp.int32)
    i = jnp.int32(0x5F3759DF) - lax.shift_right_logical(i, 1)
    y = lax.bitcast_convert_type(i, jnp.float32)
    half_x = jnp.float32(0.5) * x
    for _ in range(3):
        y = y * (jnp.float32(1.5) - half_x * y * y)
    return y


def _make_sc_kernel():
    mesh = plsc.VectorSubcoreMesh(core_axis_name="c", subcore_axis_name="s")

    @functools.partial(
        pl.kernel,
        mesh=mesh,
        compiler_params=pltpu.CompilerParams(needs_layout_passes=False),
        out_type=jax.ShapeDtypeStruct((B,), jnp.float32),
        scratch_types=[
            pltpu.VMEM((ROWS_PER_W,), jnp.int32),
            pltpu.VMEM((NCHUNK, CHUNK, D), jnp.float32),
            pltpu.VMEM((NBUF, CHUNK, D), jnp.float32),
            pltpu.VMEM((ROWS_PER_W,), jnp.float32),
            pltpu.SemaphoreType.DMA((NCHUNK,)),
            pltpu.SemaphoreType.DMA((NCHUNK,)),
            pltpu.SemaphoreType.DMA((NBUF,)),
            pltpu.SemaphoreType.DMA,
            pltpu.SemaphoreType.DMA,
        ],
    )
    def sc_kernel(src_hbm, ids_hbm, dst_hbm, table_hbm, out_hbm,
                  idx_v, acc_v, dst_v, out_v,
                  ssem, gsem, dsem, idx_sem, out_sem):
        wid = lax.axis_index("s") * NC + lax.axis_index("c")
        base = wid * ROWS_PER_W

        def start_dst(c, b):
            pltpu.async_copy(dst_hbm.at[pl.ds(base + c * CHUNK, CHUNK)],
                             dst_v.at[b], dsem.at[b])

        idx_desc = pltpu.async_copy(
            ids_hbm.at[pl.ds(base, ROWS_PER_W)], idx_v, idx_sem
        )
        for c in range(NCHUNK):
            pltpu.async_copy(src_hbm.at[pl.ds(base + c * CHUNK, CHUNK)],
                             acc_v.at[c], ssem.at[c])
        start_dst(0, 0)
        start_dst(1, 1)
        idx_desc.wait()

        def issue_body(c, _):
            pltpu.make_async_copy(src_hbm.at[pl.ds(base + c * CHUNK, CHUNK)],
                                  acc_v.at[c], ssem.at[c]).wait()
            pltpu.async_copy(
                table_hbm.at[idx_v.at[pl.ds(c * CHUNK, CHUNK)]],
                acc_v.at[c], gsem.at[c], add=True,
            )
            return 0

        lax.fori_loop(0, NCHUNK, issue_body, 0)

        lane = jnp.arange(L, dtype=jnp.int32)

        def chunk_body(c, _):
            bsel = c % NBUF
            pltpu.make_async_copy(dst_hbm.at[pl.ds(base + c * CHUNK, CHUNK)],
                                  dst_v.at[bsel], dsem.at[bsel]).wait()
            pltpu.make_async_copy(
                table_hbm.at[idx_v.at[pl.ds(c * CHUNK, CHUNK)]],
                acc_v.at[c], gsem.at[c],
            ).wait()
            av, dv = acc_v.at[c], dst_v.at[bsel]

            def group_body(g, _):
                rows = lane + g * L

                def body(j, acc):
                    cols = (lane + j) & jnp.int32(D - 1)
                    a = plsc.load_gather(av, [rows, cols])
                    d = plsc.load_gather(dv, [rows, cols])
                    t = a - d
                    return acc + t * t

                acc = lax.fori_loop(0, D, body, jnp.zeros((L,), jnp.float32),
                                    unroll=4)
                x = jnp.maximum(acc, jnp.float32(1e-30))
                out_v[pl.ds(c * CHUNK + g * L, L)] = -(x * _rsqrt_newton(x))
                return 0

            lax.fori_loop(0, CHUNK // L, group_body, 0)

            @pl.when(c + NBUF < NCHUNK)
            def _():
                start_dst(c + NBUF, bsel)

            return 0

        lax.fori_loop(0, NCHUNK, chunk_body, 0)
        pltpu.async_copy(
            out_v, out_hbm.at[pl.ds(base, ROWS_PER_W)], out_sem
        ).wait()

    return sc_kernel


_SC_KERNEL = _make_sc_kernel()


def kernel(src_emb, rel_ids, dst_emb, rel_table):
    return _SC_KERNEL(src_emb, rel_ids.astype(jnp.int32), dst_emb, rel_table)


# speedup vs baseline: 1.0132x; 1.0132x over previous
import functools

import jax
import jax.numpy as jnp
from jax import lax
from jax.experimental import pallas as pl
from jax.experimental.pallas import tpu as pltpu
from jax.experimental.pallas import tpu_sc as plsc

B = 16384
D = 128
L = 16
NC = 2
NS = 16
NW = NC * NS
ROWS_PER_W = B // NW
CHUNK = 64
NCHUNK = ROWS_PER_W // CHUNK
NBUF = 2


def _rsqrt_newton(x):
    i = lax.bitcast_convert_type(x, jnp.int32)
    i = jnp.int32(0x5F3759DF) - lax.shift_right_logical(i, 1)
    y = lax.bitcast_convert_type(i, jnp.float32)
    half_x = jnp.float32(0.5) * x
    for _ in range(3):
        y = y * (jnp.float32(1.5) - half_x * y * y)
    return y


def _make_sc_kernel():
    mesh = plsc.VectorSubcoreMesh(core_axis_name="c", subcore_axis_name="s")

    @functools.partial(
        pl.kernel,
        mesh=mesh,
        compiler_params=pltpu.CompilerParams(needs_layout_passes=False),
        out_type=jax.ShapeDtypeStruct((B,), jnp.float32),
        scratch_types=[
            pltpu.VMEM((ROWS_PER_W,), jnp.int32),
            pltpu.VMEM((NCHUNK, CHUNK, D), jnp.float32),
            pltpu.VMEM((NBUF, CHUNK, D), jnp.float32),
            pltpu.VMEM((ROWS_PER_W,), jnp.float32),
            pltpu.SemaphoreType.DMA((NCHUNK,)),
            pltpu.SemaphoreType.DMA((NCHUNK,)),
            pltpu.SemaphoreType.DMA((NBUF,)),
            pltpu.SemaphoreType.DMA,
            pltpu.SemaphoreType.DMA,
        ],
    )
    def sc_kernel(src_hbm, ids_hbm, dst_hbm, table_hbm, out_hbm,
                  idx_v, acc_v, dst_v, out_v,
                  ssem, gsem, dsem, idx_sem, out_sem):
        wid = lax.axis_index("s") * NC + lax.axis_index("c")
        base = wid * ROWS_PER_W

        def start_dst(c, b):
            pltpu.async_copy(dst_hbm.at[pl.ds(base + c * CHUNK, CHUNK)],
                             dst_v.at[b], dsem.at[b])

        idx_desc = pltpu.async_copy(
            ids_hbm.at[pl.ds(base, ROWS_PER_W)], idx_v, idx_sem
        )
        for c in range(NCHUNK):
            pltpu.async_copy(src_hbm.at[pl.ds(base + c * CHUNK, CHUNK)],
                             acc_v.at[c], ssem.at[c])
        start_dst(0, 0)
        start_dst(1, 1)
        idx_desc.wait()

        def issue_body(c, _):
            pltpu.make_async_copy(src_hbm.at[pl.ds(base + c * CHUNK, CHUNK)],
                                  acc_v.at[c], ssem.at[c]).wait()
            pltpu.async_copy(
                table_hbm.at[idx_v.at[pl.ds(c * CHUNK, CHUNK)]],
                acc_v.at[c], gsem.at[c], add=True,
            )
            return 0

        lax.fori_loop(0, NCHUNK, issue_body, 0)

        lane = jnp.arange(L, dtype=jnp.int32)

        def chunk_body(c, _):
            bsel = c % NBUF
            pltpu.make_async_copy(dst_hbm.at[pl.ds(base + c * CHUNK, CHUNK)],
                                  dst_v.at[bsel], dsem.at[bsel]).wait()
            pltpu.make_async_copy(
                table_hbm.at[idx_v.at[pl.ds(c * CHUNK, CHUNK)]],
                acc_v.at[c], gsem.at[c],
            ).wait()
            av, dv = acc_v.at[c], dst_v.at[bsel]

            def group_body(g, _):
                rows = lane + g * L

                def body(j, acc):
                    cols = (lane + j) & jnp.int32(D - 1)
                    a = plsc.load_gather(av, [rows, cols])
                    d = plsc.load_gather(dv, [rows, cols])
                    t = a - d
                    return acc + t * t

                acc = lax.fori_loop(0, D, body, jnp.zeros((L,), jnp.float32),
                                    unroll=8)
                x = jnp.maximum(acc, jnp.float32(1e-30))
                out_v[pl.ds(c * CHUNK + g * L, L)] = -(x * _rsqrt_newton(x))
                return 0

            lax.fori_loop(0, CHUNK // L, group_body, 0)

            @pl.when(c + NBUF < NCHUNK)
            def _():
                start_dst(c + NBUF, bsel)

            return 0

        lax.fori_loop(0, NCHUNK, chunk_body, 0)
        pltpu.async_copy(
            out_v, out_hbm.at[pl.ds(base, ROWS_PER_W)], out_sem
        ).wait()

    return sc_kernel


_SC_KERNEL = _make_sc_kernel()


def kernel(src_emb, rel_ids, dst_emb, rel_table):
    return _SC_KERNEL(src_emb, rel_ids.astype(jnp.int32), dst_emb, rel_table)
